# baseline (speedup 1.0000x reference)
import functools

import jax
import jax.numpy as jnp
import numpy as np
from jax import lax
from jax.experimental import pallas as pl
from jax.experimental.pallas import tpu as pltpu
from jax.experimental.pallas import tpu_sc as plsc

NC = 2
NS = 16
NW = NC * NS
L = 16

BLK = 80


def _worker_id():
    return lax.axis_index("s") * NC + lax.axis_index("c")


def _make_gather_x(vocab, n, hid):
    nblk = n // BLK
    mesh = plsc.VectorSubcoreMesh(core_axis_name="c", subcore_axis_name="s")

    @functools.partial(
        pl.kernel,
        mesh=mesh,
        compiler_params=pltpu.CompilerParams(needs_layout_passes=False),
        out_type=jax.ShapeDtypeStruct((n, hid), jnp.float32),
        scratch_types=[
            pltpu.VMEM((BLK,), jnp.int32),
            pltpu.VMEM((BLK, hid), jnp.float32),
            pltpu.SemaphoreType.DMA,
        ],
    )
    def gather_x(emb_hbm, feats_hbm, x_hbm, fidx_v, rows_v, sem):
        wid = _worker_id()
        nloop = (nblk + NW - 1) // NW

        def body(j, carry):
            blk = j * NW + wid

            @pl.when(blk < nblk)
            def _():
                base = blk * BLK
                pltpu.sync_copy(feats_hbm.at[pl.ds(base, BLK)], fidx_v)
                pltpu.async_copy(emb_hbm.at[fidx_v], rows_v, sem).wait()
                pltpu.sync_copy(rows_v, x_hbm.at[pl.ds(base, BLK)])

            return carry

        lax.fori_loop(0, nloop, body, 0)

    return gather_x


def _matmul_h(x, W, b):
    n, hid = x.shape
    rows = 1000
    grid = n // rows

    def body(x_ref, w_ref, b_ref, h_ref):
        h_ref[...] = (
            jnp.dot(x_ref[...], w_ref[...], preferred_element_type=jnp.float32)
            + b_ref[...]
        ).astype(jnp.bfloat16)

    return pl.pallas_call(
        body,
        grid=(grid,),
        in_specs=[
            pl.BlockSpec((rows, hid), lambda i: (i, 0)),
            pl.BlockSpec((hid, hid), lambda i: (0, 0)),
            pl.BlockSpec((1, hid), lambda i: (0, 0)),
        ],
        out_specs=pl.BlockSpec((rows, hid), lambda i: (i, 0)),
        out_shape=jax.ShapeDtypeStruct((n, hid), jnp.bfloat16),
    )(x, W, b.reshape(1, hid))


def _make_edge_phase(n, e, hid, npad):
    e_per_w = e // NW
    nblk = e_per_w // BLK
    mesh = plsc.VectorSubcoreMesh(core_axis_name="c", subcore_axis_name="s")

    @functools.partial(
        pl.kernel,
        mesh=mesh,
        compiler_params=pltpu.CompilerParams(
            needs_layout_passes=False, use_tc_tiling_on_sc=False),
        out_type=[
            jax.ShapeDtypeStruct((NW, npad), jnp.float32),
            jax.ShapeDtypeStruct((NW, npad), jnp.float32),
        ],
        scratch_types=[
            pltpu.VMEM((e_per_w,), jnp.int32),
            pltpu.VMEM((e_per_w,), jnp.int32),
            pltpu.VMEM((2, BLK, hid), jnp.bfloat16),
            pltpu.VMEM((2, BLK, hid), jnp.bfloat16),
            pltpu.VMEM_SHARED((n, hid), jnp.bfloat16),
            pltpu.VMEM((hid,), jnp.float32),
            pltpu.VMEM((hid,), jnp.float32),
            pltpu.VMEM((npad,), jnp.float32),
            pltpu.VMEM((npad,), jnp.float32),
            pltpu.SemaphoreType.DMA,
            pltpu.SemaphoreType.DMA,
            pltpu.SemaphoreType.DMA,
            pltpu.SemaphoreType.DMA,
        ],
    )
    def edge_phase(
        h_hbm, src_hbm, dst_hbm, attn_hbm, regw_hbm,
        nump_hbm, denp_hbm,
        sidx_v, didx_v, srows_v, drows_v, hsh_v, attn_v, regw_v, num_v, den_v,
        sem_s0, sem_s1, sem_d0, sem_d1,
    ):
        wid = _worker_id()
        sid = lax.axis_index("s")
        pltpu.sync_copy(attn_hbm, attn_v)
        pltpu.sync_copy(regw_hbm, regw_v)
        nrblk = n // BLK

        def stage_body(j, carry):
            rb = j * NS + sid

            @pl.when(rb < nrblk)
            def _():
                pltpu.sync_copy(h_hbm.at[pl.ds(rb * BLK, BLK)],
                                hsh_v.at[pl.ds(rb * BLK, BLK)])

            return carry

        lax.fori_loop(0, (nrblk + NS - 1) // NS, stage_body, 0)
        plsc.subcore_barrier()
        ebase = wid * e_per_w
        pltpu.sync_copy(src_hbm.at[pl.ds(ebase, e_per_w)], sidx_v)
        pltpu.sync_copy(dst_hbm.at[pl.ds(ebase, e_per_w)], didx_v)

        zero16 = jnp.zeros((L,), jnp.float32)

        def zi(i, carry):
            num_v[pl.ds(i * L, L)] = zero16
            den_v[pl.ds(i * L, L)] = zero16
            return carry

        lax.fori_loop(0, npad // L, zi, 0)

        sem_s = [sem_s0, sem_s1]
        sem_d = [sem_d0, sem_d1]

        def start_fetch(bi, slot):
            pltpu.async_copy(
                hsh_v.at[sidx_v.at[pl.ds(bi * BLK, BLK)]],
                srows_v.at[slot], sem_s[slot])
            pltpu.async_copy(
                hsh_v.at[didx_v.at[pl.ds(bi * BLK, BLK)]],
                drows_v.at[slot], sem_d[slot])

        def wait_fetch(bi, slot):
            pltpu.make_async_copy(
                hsh_v.at[sidx_v.at[pl.ds(bi * BLK, BLK)]],
                srows_v.at[slot], sem_s[slot]).wait()
            pltpu.make_async_copy(
                hsh_v.at[didx_v.at[pl.ds(bi * BLK, BLK)]],
                drows_v.at[slot], sem_d[slot]).wait()

        nchunk = hid // L
        eunroll = 4

        def compute(bi, slot):
            srows = srows_v.at[slot]
            drows = drows_v.at[slot]
            coeffs = tuple(
                [attn_v[pl.ds(c * L, L)] for c in range(nchunk)]
                + [regw_v[pl.ds(c * L, L)] for c in range(nchunk)]
            )

            acs = coeffs[:nchunk]
            rcs = coeffs[nchunk:]
            lane = jnp.arange(L, dtype=jnp.int32)

            def group_body(g, gc):
                def quad_body(t, c2):
                    lvec, rvec = c2
                    for u in range(eunroll):
                        e = g * L + t * eunroll + u
                        a_lo = a_hi = r_lo = r_hi = zero16
                        for c2i in range(nchunk // 2):
                            sk32 = srows[e, pl.ds(c2i * 2 * L, 2 * L)]
                            dk32 = drows[e, pl.ds(c2i * 2 * L, 2 * L)]
                            se, so = plsc.unpack(
                                sk32, format=plsc.PackFormat.INTERLEAVED)
                            de, do_ = plsc.unpack(
                                dk32, format=plsc.PackFormat.INTERLEAVED)
                            for sk, dk, c in (
                                (se, de, c2i * 2),
                                (so, do_, c2i * 2 + 1),
                            ):
                                pre = sk + dk
                                act = jnp.maximum(pre, 0.2 * pre)
                                if c % 2 == 0:
                                    a_lo = a_lo + acs[c] * act
                                    r_lo = r_lo + rcs[c] * sk
                                else:
                                    a_hi = a_hi + acs[c] * act
                                    r_hi = r_hi + rcs[c] * sk
                        lsum = jnp.sum(a_lo + a_hi)
                        rsum = jnp.sum(r_lo + r_hi)
                        m = lane == (t * eunroll + u)
                        lvec = jnp.where(m, lsum, lvec)
                        rvec = jnp.where(m, rsum, rvec)
                    return (lvec, rvec)

                lvec, rvec = lax.fori_loop(
                    0, L // eunroll, quad_body, (zero16, zero16))
                ex = jnp.exp(lvec)
                dstv = didx_v[pl.ds(bi * BLK + g * L, L)]
                plsc.addupdate_scatter(den_v, [dstv], ex)
                plsc.addupdate_scatter(num_v, [dstv], ex * rvec)
                return gc

            lax.fori_loop(0, BLK // L, group_body, 0)

        assert nblk % 2 == 1 and nblk >= 3
        start_fetch(0, 0)

        def pair_body(i, carry):
            b0 = i * 2
            start_fetch(b0 + 1, 1)
            wait_fetch(b0, 0)
            compute(b0, 0)
            start_fetch(b0 + 2, 0)
            wait_fetch(b0 + 1, 1)
            compute(b0 + 1, 1)
            return carry

        lax.fori_loop(0, (nblk - 1) // 2, pair_body, 0)
        wait_fetch(nblk - 1, 0)
        compute(nblk - 1, 0)

        pltpu.sync_copy(num_v, nump_hbm.at[wid])
        pltpu.sync_copy(den_v, denp_hbm.at[wid])

    return edge_phase


def _finish(nump, denp):
    nw, npad = nump.shape

    def body(np_ref, dp_ref, o_ref):
        num = jnp.sum(np_ref[...], axis=0, keepdims=True)
        den = jnp.sum(dp_ref[...], axis=0, keepdims=True)
        good = den > 0.0
        val = jnp.where(good, num / jnp.where(good, den, 1.0), 0.0)
        o_ref[...] = jnp.sum(val).reshape(1, 1)

    return pl.pallas_call(
        body,
        out_shape=jax.ShapeDtypeStruct((1, 1), jnp.float32),
    )(nump, denp)


def kernel(feats, edge_index, emb, W, b, attn, reg_W):
    vocab, hid = emb.shape
    n = feats.shape[0]
    e = edge_index.shape[1]
    npad = ((n + 127) // 128) * 128
    assert n % BLK == 0 and e % (NW * BLK) == 0 and hid == 128

    src = edge_index[0]
    dst = edge_index[1]
    regw = reg_W[:, 0]
    perm = np.concatenate(
        [np.concatenate([np.arange(c * 32, (c + 1) * 32, 2),
                         np.arange(c * 32 + 1, (c + 1) * 32, 2)])
         for c in range(hid // 32)])
    attn_r = attn[perm]
    regw_r = regw[perm]

    x = _make_gather_x(vocab, n, hid)(emb, feats)
    h = _matmul_h(x, W, b)
    nump, denp = _make_edge_phase(n, e, hid, npad)(h, src, dst, attn_r, regw_r)
    return _finish(nump, denp)


# speedup vs baseline: 1.5345x; 1.5345x over previous
import functools

import jax
import jax.numpy as jnp
import numpy as np
from jax import lax
from jax.experimental import pallas as pl
from jax.experimental.pallas import tpu as pltpu
from jax.experimental.pallas import tpu_sc as plsc

NC = 2
NS = 16
NW = NC * NS
L = 16

BLK = 80


def _worker_id():
    return lax.axis_index("s") * NC + lax.axis_index("c")


def _make_gather_x(vocab, n, hid):
    nblk = n // BLK
    mesh = plsc.VectorSubcoreMesh(core_axis_name="c", subcore_axis_name="s")

    @functools.partial(
        pl.kernel,
        mesh=mesh,
        compiler_params=pltpu.CompilerParams(needs_layout_passes=False),
        out_type=jax.ShapeDtypeStruct((n, hid), jnp.float32),
        scratch_types=[
            pltpu.VMEM((BLK,), jnp.int32),
            pltpu.VMEM((BLK, hid), jnp.float32),
            pltpu.SemaphoreType.DMA,
        ],
    )
    def gather_x(emb_hbm, feats_hbm, x_hbm, fidx_v, rows_v, sem):
        wid = _worker_id()
        nloop = (nblk + NW - 1) // NW

        def body(j, carry):
            blk = j * NW + wid

            @pl.when(blk < nblk)
            def _():
                base = blk * BLK
                pltpu.sync_copy(feats_hbm.at[pl.ds(base, BLK)], fidx_v)
                pltpu.async_copy(emb_hbm.at[fidx_v], rows_v, sem).wait()
                pltpu.sync_copy(rows_v, x_hbm.at[pl.ds(base, BLK)])

            return carry

        lax.fori_loop(0, nloop, body, 0)

    return gather_x


def _matmul_h(x, W, b):
    n, hid = x.shape
    rows = 1000
    grid = n // rows

    def body(x_ref, w_ref, b_ref, h_ref):
        h_ref[...] = (
            jnp.dot(x_ref[...], w_ref[...], preferred_element_type=jnp.float32)
            + b_ref[...]
        )

    return pl.pallas_call(
        body,
        grid=(grid,),
        in_specs=[
            pl.BlockSpec((rows, hid), lambda i: (i, 0)),
            pl.BlockSpec((hid, hid), lambda i: (0, 0)),
            pl.BlockSpec((1, hid), lambda i: (0, 0)),
        ],
        out_specs=pl.BlockSpec((rows, hid), lambda i: (i, 0)),
        out_shape=jax.ShapeDtypeStruct((n, hid), jnp.float32),
    )(x, W, b.reshape(1, hid))


def _make_edge_phase(n, e, hid, npad):
    e_per_w = e // NW
    nblk = e_per_w // BLK
    mesh = plsc.VectorSubcoreMesh(core_axis_name="c", subcore_axis_name="s")

    @functools.partial(
        pl.kernel,
        mesh=mesh,
        compiler_params=pltpu.CompilerParams(
            needs_layout_passes=False, use_tc_tiling_on_sc=False),
        out_type=[
            jax.ShapeDtypeStruct((NW, npad), jnp.float32),
            jax.ShapeDtypeStruct((NW, npad), jnp.float32),
        ],
        scratch_types=[
            pltpu.VMEM((e_per_w,), jnp.int32),
            pltpu.VMEM((e_per_w,), jnp.int32),
            pltpu.VMEM((2, BLK, hid), jnp.float32),
            pltpu.VMEM((2, BLK, hid), jnp.float32),
            pltpu.VMEM((hid,), jnp.float32),
            pltpu.VMEM((hid,), jnp.float32),
            pltpu.VMEM((npad,), jnp.float32),
            pltpu.VMEM((npad,), jnp.float32),
            pltpu.SemaphoreType.DMA,
            pltpu.SemaphoreType.DMA,
            pltpu.SemaphoreType.DMA,
            pltpu.SemaphoreType.DMA,
        ],
    )
    def edge_phase(
        h_hbm, src_hbm, dst_hbm, attn_hbm, regw_hbm,
        nump_hbm, denp_hbm,
        sidx_v, didx_v, srows_v, drows_v, attn_v, regw_v, num_v, den_v,
        sem_s0, sem_s1, sem_d0, sem_d1,
    ):
        wid = _worker_id()
        pltpu.sync_copy(attn_hbm, attn_v)
        pltpu.sync_copy(regw_hbm, regw_v)
        ebase = wid * e_per_w
        pltpu.sync_copy(src_hbm.at[pl.ds(ebase, e_per_w)], sidx_v)
        pltpu.sync_copy(dst_hbm.at[pl.ds(ebase, e_per_w)], didx_v)

        zero16 = jnp.zeros((L,), jnp.float32)

        def zi(i, carry):
            num_v[pl.ds(i * L, L)] = zero16
            den_v[pl.ds(i * L, L)] = zero16
            return carry

        lax.fori_loop(0, npad // L, zi, 0)

        sem_s = [sem_s0, sem_s1]
        sem_d = [sem_d0, sem_d1]

        def start_fetch(bi, slot):
            pltpu.async_copy(
                h_hbm.at[sidx_v.at[pl.ds(bi * BLK, BLK)]],
                srows_v.at[slot], sem_s[slot])
            pltpu.async_copy(
                h_hbm.at[didx_v.at[pl.ds(bi * BLK, BLK)]],
                drows_v.at[slot], sem_d[slot])

        def wait_fetch(bi, slot):
            pltpu.make_async_copy(
                h_hbm.at[sidx_v.at[pl.ds(bi * BLK, BLK)]],
                srows_v.at[slot], sem_s[slot]).wait()
            pltpu.make_async_copy(
                h_hbm.at[didx_v.at[pl.ds(bi * BLK, BLK)]],
                drows_v.at[slot], sem_d[slot]).wait()

        nchunk = hid // L
        eunroll = 16

        def compute(bi, slot):
            srows = srows_v.at[slot]
            drows = drows_v.at[slot]
            coeffs = tuple(
                [attn_v[pl.ds(c * L, L)] for c in range(nchunk)]
                + [regw_v[pl.ds(c * L, L)] for c in range(nchunk)]
            )

            acs = coeffs[:nchunk]
            rcs = coeffs[nchunk:]
            lane = jnp.arange(L, dtype=jnp.int32)

            def group_body(g, gc):
                def quad_body(t, c2):
                    lvec, rvec = c2
                    for u in range(eunroll):
                        e = g * L + t * eunroll + u
                        a_lo = a_hi = r_lo = r_hi = zero16
                        for c in range(nchunk):
                            sk = srows[e, pl.ds(c * L, L)]
                            dk = drows[e, pl.ds(c * L, L)]
                            pre = sk + dk
                            act = jnp.maximum(pre, 0.2 * pre)
                            if c % 2 == 0:
                                a_lo = a_lo + acs[c] * act
                                r_lo = r_lo + rcs[c] * sk
                            else:
                                a_hi = a_hi + acs[c] * act
                                r_hi = r_hi + rcs[c] * sk
                        lsum = jnp.sum(a_lo + a_hi)
                        rsum = jnp.sum(r_lo + r_hi)
                        m = lane == (t * eunroll + u)
                        lvec = jnp.where(m, lsum, lvec)
                        rvec = jnp.where(m, rsum, rvec)
                    return (lvec, rvec)

                lvec, rvec = lax.fori_loop(
                    0, L // eunroll, quad_body, (zero16, zero16))
                ex = jnp.exp(lvec)
                dstv = didx_v[pl.ds(bi * BLK + g * L, L)]
                plsc.addupdate_scatter(den_v, [dstv], ex)
                plsc.addupdate_scatter(num_v, [dstv], ex * rvec)
                return gc

            lax.fori_loop(0, BLK // L, group_body, 0)

        assert nblk % 2 == 1 and nblk >= 3
        start_fetch(0, 0)

        def pair_body(i, carry):
            b0 = i * 2
            start_fetch(b0 + 1, 1)
            wait_fetch(b0, 0)
            compute(b0, 0)
            start_fetch(b0 + 2, 0)
            wait_fetch(b0 + 1, 1)
            compute(b0 + 1, 1)
            return carry

        lax.fori_loop(0, (nblk - 1) // 2, pair_body, 0)
        wait_fetch(nblk - 1, 0)
        compute(nblk - 1, 0)

        pltpu.sync_copy(num_v, nump_hbm.at[wid])
        pltpu.sync_copy(den_v, denp_hbm.at[wid])

    return edge_phase


def _finish(nump, denp):
    nw, npad = nump.shape

    def body(np_ref, dp_ref, o_ref):
        num = jnp.sum(np_ref[...], axis=0, keepdims=True)
        den = jnp.sum(dp_ref[...], axis=0, keepdims=True)
        good = den > 0.0
        val = jnp.where(good, num / jnp.where(good, den, 1.0), 0.0)
        o_ref[...] = jnp.sum(val).reshape(1, 1)

    return pl.pallas_call(
        body,
        out_shape=jax.ShapeDtypeStruct((1, 1), jnp.float32),
    )(nump, denp)


def kernel(feats, edge_index, emb, W, b, attn, reg_W):
    vocab, hid = emb.shape
    n = feats.shape[0]
    e = edge_index.shape[1]
    npad = ((n + 127) // 128) * 128
    assert n % BLK == 0 and e % (NW * BLK) == 0 and hid == 128

    src = edge_index[0]
    dst = edge_index[1]
    regw = reg_W[:, 0]

    x = _make_gather_x(vocab, n, hid)(emb, feats)
    h = _matmul_h(x, W, b)
    nump, denp = _make_edge_phase(n, e, hid, npad)(h, src, dst, attn, regw)
    return _finish(nump, denp)
